# baseline (speedup 1.0000x reference)
import jax
import jax.numpy as jnp
from jax import lax
from jax.experimental import pallas as pl
from jax.experimental.pallas import tpu as pltpu
from jax.experimental.pallas import tpu_sc as plsc

_B = 16384
_D = 64
_NC = 2
_NS = 16
_NW = _NC * _NS
_BPW = _B // _NW
_CHUNK = 128
_NCHUNK = _BPW // _CHUNK
_GROUPS = _BPW // 16


def _body(entT_hbm, relT_hbm, heads_hbm, rels_hbm, tails_hbm, out_hbm,
          hidx, ridx, tidx, e1_v, r_v, e2_v, out_v, sem):
    wid = lax.axis_index("s") * _NC + lax.axis_index("c")
    base = wid * _BPW

    pltpu.sync_copy(heads_hbm.at[pl.ds(base, _BPW)], hidx)
    pltpu.sync_copy(rels_hbm.at[pl.ds(base, _BPW)], ridx)
    pltpu.sync_copy(tails_hbm.at[pl.ds(base, _BPW)], tidx)

    copies = []
    for d in range(_D):
        for q in range(_NCHUNK):
            isl = pl.ds(q * _CHUNK, _CHUNK)
            copies.append(pltpu.async_copy(
                entT_hbm.at[d].at[hidx.at[isl]], e1_v.at[d, isl], sem))
            copies.append(pltpu.async_copy(
                relT_hbm.at[d].at[ridx.at[isl]], r_v.at[d, isl], sem))
            copies.append(pltpu.async_copy(
                entT_hbm.at[d].at[tidx.at[isl]], e2_v.at[d, isl], sem))
    for c in copies:
        c.wait()

    def group(g, carry):
        col0 = g * 16
        sl = pl.ds(col0, 16)
        s = (e1_v[0, sl] * r_v[0, sl]) * e2_v[0, sl]
        for d in range(1, _D):
            s = s + (e1_v[d, sl] * r_v[d, sl]) * e2_v[d, sl]
        out_v[sl] = 1.0 / (1.0 + jnp.exp(-s))
        return carry

    lax.fori_loop(0, _GROUPS, group, 0)
    pltpu.sync_copy(out_v, out_hbm.at[pl.ds(base, _BPW)])


def kernel(entity_embedding, relation_embedding, heads, relations, tails):
    mesh = plsc.VectorSubcoreMesh(core_axis_name="c", subcore_axis_name="s")
    run = pl.kernel(
        _body,
        out_type=jax.ShapeDtypeStruct((_B,), jnp.float32),
        mesh=mesh,
        compiler_params=pltpu.CompilerParams(use_tc_tiling_on_sc=False),
        scratch_types=[
            pltpu.VMEM((_BPW,), jnp.int32),
            pltpu.VMEM((_BPW,), jnp.int32),
            pltpu.VMEM((_BPW,), jnp.int32),
            pltpu.VMEM((_D, _BPW), jnp.float32),
            pltpu.VMEM((_D, _BPW), jnp.float32),
            pltpu.VMEM((_D, _BPW), jnp.float32),
            pltpu.VMEM((_BPW,), jnp.float32),
            pltpu.SemaphoreType.DMA,
        ],
    )
    return run(entity_embedding.T, relation_embedding.T,
               heads.astype(jnp.int32), relations.astype(jnp.int32),
               tails.astype(jnp.int32))


# speedup vs baseline: 11.9566x; 11.9566x over previous
import jax
import jax.numpy as jnp
from jax import lax
from jax.experimental import pallas as pl
from jax.experimental.pallas import tpu as pltpu
from jax.experimental.pallas import tpu_sc as plsc

_B = 16384
_D = 64
_NC = 2
_NS = 16
_NW = _NC * _NS
_BPW = _B // _NW
_CH = 16
_NCHUNK = _BPW // _CH


def _fire(ent_hbm, rel_hbm, hvec, rvec, tvec, e1b, rb, e2b, sem):
    copies = []
    for j in range(_CH):
        h8 = pl.multiple_of((hvec[j] >> 3) * 8, 8)
        r8 = pl.multiple_of((rvec[j] >> 3) * 8, 8)
        t8 = pl.multiple_of((tvec[j] >> 3) * 8, 8)
        copies.append(pltpu.async_copy(
            ent_hbm.at[pl.ds(h8, 8), :], e1b.at[j], sem))
        copies.append(pltpu.async_copy(
            rel_hbm.at[pl.ds(r8, 8), :], rb.at[j], sem))
        copies.append(pltpu.async_copy(
            ent_hbm.at[pl.ds(t8, 8), :], e2b.at[j], sem))
    return copies


def _body(ent_hbm, rel_hbm, heads_hbm, rels_hbm, tails_hbm, out_hbm,
          hidx, ridx, tidx, e1b2, rb2, e2b2, out_v, sem):
    wid = lax.axis_index("s") * _NC + lax.axis_index("c")
    base = wid * _BPW

    pltpu.sync_copy(heads_hbm.at[pl.ds(base, _BPW)], hidx)
    pltpu.sync_copy(rels_hbm.at[pl.ds(base, _BPW)], ridx)
    pltpu.sync_copy(tails_hbm.at[pl.ds(base, _BPW)], tidx)

    lanes16 = lax.iota(jnp.int32, 16)
    bfly = [jnp.bitwise_xor(lanes16, sh) for sh in (8, 4, 2, 1)]
    dnums = lax.GatherDimensionNumbers(
        offset_dims=(), collapsed_slice_dims=(0,), start_index_map=(0,))

    def shuffle(v, idx):
        return lax.gather(v, idx[:, None], dnums, slice_sizes=(1,),
                          mode=lax.GatherScatterMode.PROMISE_IN_BOUNDS)

    def lanesum(v):
        for idx in bfly:
            v = v + shuffle(v, idx)
        return v

    def idx_chunk(k):
        sl = pl.ds(k * _CH, _CH)
        return hidx[sl], ridx[sl], tidx[sl]

    def drain(slot):
        dummy = ent_hbm.at[pl.ds(0, 8), :]
        for j in range(_CH):
            pltpu.make_async_copy(dummy, e1b2.at[slot, j], sem).wait()
            pltpu.make_async_copy(dummy, rb2.at[slot, j], sem).wait()
            pltpu.make_async_copy(dummy, e2b2.at[slot, j], sem).wait()

    def fire_chunk(k, slot):
        hv, rv, tv = idx_chunk(k)
        _fire(ent_hbm, rel_hbm, hv, rv, tv,
              e1b2.at[slot], rb2.at[slot], e2b2.at[slot], sem)

    def compute_chunk(k, slot):
        hvec, rvec, tvec = idx_chunk(k)
        s = jnp.zeros((16,), jnp.float32)
        for j in range(_CH):
            hs = hvec[j] & 7
            rs = rvec[j] & 7
            ts = tvec[j] & 7
            acc = (e1b2[slot, j, hs, pl.ds(0, 16)]
                   * rb2[slot, j, rs, pl.ds(0, 16)]) \
                * e2b2[slot, j, ts, pl.ds(0, 16)]
            for c in range(1, _D // 16):
                acc = acc + (e1b2[slot, j, hs, pl.ds(c * 16, 16)]
                             * rb2[slot, j, rs, pl.ds(c * 16, 16)]) \
                    * e2b2[slot, j, ts, pl.ds(c * 16, 16)]
            s = jnp.where(lanes16 == j, lanesum(acc), s)
        out_v[pl.ds(k * _CH, _CH)] = 1.0 / (1.0 + jnp.exp(-s))

    fire_chunk(0, 0)

    def step(i, carry):
        a = i * 2
        fire_chunk(a + 1, 1)
        drain(0)
        compute_chunk(a, 0)
        fire_chunk(jnp.minimum(a + 2, _NCHUNK - 1), 0)
        drain(1)
        compute_chunk(a + 1, 1)
        return carry

    lax.fori_loop(0, _NCHUNK // 2, step, 0)
    drain(0)

    pltpu.sync_copy(out_v, out_hbm.at[pl.ds(base, _BPW)])


def kernel(entity_embedding, relation_embedding, heads, relations, tails):
    mesh = plsc.VectorSubcoreMesh(core_axis_name="c", subcore_axis_name="s")
    run = pl.kernel(
        _body,
        out_type=jax.ShapeDtypeStruct((_B,), jnp.float32),
        mesh=mesh,
        scratch_types=[
            pltpu.VMEM((_BPW,), jnp.int32),
            pltpu.VMEM((_BPW,), jnp.int32),
            pltpu.VMEM((_BPW,), jnp.int32),
            pltpu.VMEM((2, _CH, 8, _D), jnp.float32),
            pltpu.VMEM((2, _CH, 8, _D), jnp.float32),
            pltpu.VMEM((2, _CH, 8, _D), jnp.float32),
            pltpu.VMEM((_BPW,), jnp.float32),
            pltpu.SemaphoreType.DMA,
        ],
    )
    return run(entity_embedding, relation_embedding,
               heads.astype(jnp.int32), relations.astype(jnp.int32),
               tails.astype(jnp.int32))
